# baseline (speedup 1.0000x reference)
import jax
import jax.numpy as jnp
import numpy as np
from jax.experimental import pallas as pl
from jax.experimental.pallas import tpu as pltpu

_B, _D, _N, _P = 1024, 128, 100000, 1000
_TEMP = 0.05
_NB = 5000
_NUM_BLK = _N // _NB
_SCALE = float(np.log2(np.e)) / _TEMP


def _loss_kernel(img_ref, pids_ref, feats_ref, mpids_ref, ones_ref, out_ref,
                 nimg_ref, pos_ref, all_ref):
    i = pl.program_id(0)

    @pl.when(i == 0)
    def _init():
        img = img_ref[...]
        norm = jnp.sqrt(jnp.sum(img * img, axis=1, keepdims=True))
        nimg_ref[...] = (img * _SCALE / jnp.maximum(norm, 1e-12)
                         ).astype(jnp.bfloat16)
        pos_ref[...] = jnp.zeros_like(pos_ref)
        all_ref[...] = jnp.zeros_like(all_ref)

    feats = feats_ref[...].astype(jnp.bfloat16)
    scores = jax.lax.dot_general(
        nimg_ref[...], feats, (((1,), (1,)), ((), ())),
        preferred_element_type=jnp.float32)
    e = jnp.exp2(scores).astype(jnp.bfloat16)
    labels = pids_ref[...] == mpids_ref[0]
    masked = jnp.where(labels, e, jnp.bfloat16(0.0))
    ones = ones_ref[...]
    pos_ref[...] += jax.lax.dot_general(
        masked, ones, (((1,), (0,)), ((), ())),
        preferred_element_type=jnp.float32)
    all_ref[...] += jax.lax.dot_general(
        e, ones, (((1,), (0,)), ((), ())),
        preferred_element_type=jnp.float32)

    @pl.when(i == _NUM_BLK - 1)
    def _fini():
        loss = -jnp.log(pos_ref[...] / all_ref[...] + 1e-8)
        out_ref[...] = jnp.sum(loss).reshape(1, 1) / _B


def kernel(image_inputs, text_inputs, image_ids, pids, features, memory_pids):
    del text_inputs, image_ids
    pids2 = pids.reshape(_B, 1).astype(jnp.int16)
    mpids3 = memory_pids.reshape(_NUM_BLK, 1, _NB).astype(jnp.int16)
    ones = jnp.ones((_NB, 1), jnp.bfloat16)
    out = pl.pallas_call(
        _loss_kernel,
        grid=(_NUM_BLK,),
        in_specs=[
            pl.BlockSpec((_B, _D), lambda i: (0, 0)),
            pl.BlockSpec((_B, 1), lambda i: (0, 0)),
            pl.BlockSpec((_NB, _D), lambda i: (i, 0)),
            pl.BlockSpec((1, 1, _NB), lambda i: (i, 0, 0)),
            pl.BlockSpec((_NB, 1), lambda i: (0, 0)),
        ],
        out_specs=pl.BlockSpec((1, 1), lambda i: (0, 0)),
        out_shape=jax.ShapeDtypeStruct((1, 1), jnp.float32),
        scratch_shapes=[
            pltpu.VMEM((_B, _D), jnp.bfloat16),
            pltpu.VMEM((_B, 1), jnp.float32),
            pltpu.VMEM((_B, 1), jnp.float32),
        ],
        compiler_params=pltpu.CompilerParams(
            dimension_semantics=("arbitrary",)),
    )(image_inputs, pids2, features, mpids3, ones)
    return out[0, 0]


# speedup vs baseline: 1.7238x; 1.7238x over previous
import jax
import jax.numpy as jnp
import numpy as np
from jax.experimental import pallas as pl
from jax.experimental.pallas import tpu as pltpu

_B, _D, _N, _P = 1024, 128, 100000, 1000
_TEMP = 0.05
_NB = 5000
_NUM_BLK = _N // _NB
_CHUNKS = (0, 1280, 2560, 3840, 5000)
_SCALE = float(np.log2(np.e)) / _TEMP


def _loss_kernel(img_ref, pids_ref, feats_ref, mpids_ref, out_ref,
                 nimg_ref, pos_ref, all_ref):
    i = pl.program_id(0)

    @pl.when(i == 0)
    def _init():
        img = img_ref[...]
        norm = jnp.sqrt(jnp.sum(img * img, axis=1, keepdims=True))
        nimg_ref[...] = (img * _SCALE / jnp.maximum(norm, 1e-12)
                         ).astype(jnp.bfloat16)
        pos_ref[...] = jnp.zeros_like(pos_ref)
        all_ref[...] = jnp.zeros_like(all_ref)

    nimg = nimg_ref[...]
    pids = pids_ref[...]

    pos_acc = jnp.zeros((_B, 1), jnp.float32)
    all_acc = jnp.zeros((_B, 1), jnp.float32)
    for lo, hi in zip(_CHUNKS[:-1], _CHUNKS[1:]):
        feats = feats_ref[lo:hi, :].astype(jnp.bfloat16)
        scores = jax.lax.dot_general(
            nimg, feats, (((1,), (1,)), ((), ())),
            preferred_element_type=jnp.float32)
        e = jnp.exp2(scores)
        labels = pids == mpids_ref[0, :, lo:hi]
        pos_acc += jnp.sum(jnp.where(labels, e, 0.0), axis=1, keepdims=True)
        all_acc += jnp.sum(e, axis=1, keepdims=True)
    pos_ref[...] += pos_acc
    all_ref[...] += all_acc

    @pl.when(i == _NUM_BLK - 1)
    def _fini():
        loss = -jnp.log(pos_ref[...] / all_ref[...] + 1e-8)
        out_ref[...] = jnp.sum(loss).reshape(1, 1) / _B


def kernel(image_inputs, text_inputs, image_ids, pids, features, memory_pids):
    del text_inputs, image_ids
    pids2 = pids.reshape(_B, 1)
    mpids3 = memory_pids.reshape(_NUM_BLK, 1, _NB)
    out = pl.pallas_call(
        _loss_kernel,
        grid=(_NUM_BLK,),
        in_specs=[
            pl.BlockSpec((_B, _D), lambda i: (0, 0)),
            pl.BlockSpec((_B, 1), lambda i: (0, 0)),
            pl.BlockSpec((_NB, _D), lambda i: (i, 0)),
            pl.BlockSpec((1, 1, _NB), lambda i: (i, 0, 0)),
        ],
        out_specs=pl.BlockSpec((1, 1), lambda i: (0, 0)),
        out_shape=jax.ShapeDtypeStruct((1, 1), jnp.float32),
        scratch_shapes=[
            pltpu.VMEM((_B, _D), jnp.bfloat16),
            pltpu.VMEM((_B, 1), jnp.float32),
            pltpu.VMEM((_B, 1), jnp.float32),
        ],
        compiler_params=pltpu.CompilerParams(
            dimension_semantics=("arbitrary",)),
    )(image_inputs, pids2, features, mpids3)
    return out[0, 0]
